# baseline (speedup 1.0000x reference)
import functools

import jax
import jax.numpy as jnp
from jax import lax
from jax.experimental import pallas as pl
from jax.experimental.pallas import tpu as pltpu
from jax.experimental.pallas import tpu_sc as plsc

B, S, HID, QLR, H, D, RD, TOPK = 1, 2048, 2048, 1536, 16, 128, 64, 2048
D2 = RD // 2


_NW = 32
_RPW = S // _NW
_NV = S // 16
_BINS = 256


def _sc_sort_kernel(scores_hbm, out_hbm, fbuf, ka, va, kb, vb, hist):
    wid = lax.axis_index("s") * 2 + lax.axis_index("c")

    def do_row(i, _):
        row = wid * _RPW + i
        pltpu.sync_copy(scores_hbm.at[row], fbuf)

        def xform(j, _):
            x = fbuf[pl.ds(j * 16, 16)]
            xb = plsc.bitcast(x, jnp.int32)
            xb = jnp.where(x == 0.0, 0, xb)
            key = jnp.where(xb < 0, xb, xb ^ 0x7FFFFFFF)
            ka[pl.ds(j * 16, 16)] = key
            va[pl.ds(j * 16, 16)] = lax.iota(jnp.int32, 16) + j * 16
            return 0

        lax.fori_loop(0, _NV, xform, 0)

        for p, (sk, sv, dk, dv) in enumerate(
                [(ka, va, kb, vb), (kb, vb, ka, va)] * 2):
            shift = 8 * p

            def zero(j, _):
                hist[pl.ds(j * 16, 16)] = jnp.zeros((16,), jnp.int32)
                return 0

            lax.fori_loop(0, _BINS // 16, zero, 0)

            def count(j, _, sk=sk, shift=shift):
                key = sk[pl.ds(j * 16, 16)]
                dgt = lax.shift_right_logical(key, shift) & 0xFF
                occ, last = plsc.scan_count(dgt)
                plsc.addupdate_scatter(hist, [dgt], occ, mask=last)
                return 0

            lax.fori_loop(0, _NV, count, 0)

            carry = jnp.int32(0)
            for j in range(_BINS // 16):
                c = hist[pl.ds(j * 16, 16)]
                incl = plsc.cumsum(c)
                if isinstance(incl, tuple):
                    incl = incl[0]
                hist[pl.ds(j * 16, 16)] = incl - c + carry
                carry = carry + jnp.sum(c, axis=0)

            def permute(j, _, sk=sk, sv=sv, dk=dk, dv=dv, shift=shift):
                key = sk[pl.ds(j * 16, 16)]
                val = sv[pl.ds(j * 16, 16)]
                dgt = lax.shift_right_logical(key, shift) & 0xFF
                occ, last = plsc.scan_count(dgt)
                base = plsc.load_gather(hist, [dgt])
                rank = base + occ - 1
                plsc.store_scatter(dk, [rank], key)
                plsc.store_scatter(dv, [rank], val)
                plsc.store_scatter(hist, [dgt], base + occ, mask=last)
                return 0

            lax.fori_loop(0, _NV, permute, 0)

        pltpu.sync_copy(va, out_hbm.at[row])
        return 0

    lax.fori_loop(0, _RPW, do_row, 0)


@jax.jit
def _sc_argsort_desc(scores):
    mesh = plsc.VectorSubcoreMesh(core_axis_name="c", subcore_axis_name="s")
    f = pl.kernel(
        _sc_sort_kernel,
        mesh=mesh,
        compiler_params=pltpu.CompilerParams(needs_layout_passes=False),
        out_type=jax.ShapeDtypeStruct((S, S), jnp.int32),
        scratch_types=[
            pltpu.VMEM((S,), jnp.float32),
            pltpu.VMEM((S,), jnp.int32),
            pltpu.VMEM((S,), jnp.int32),
            pltpu.VMEM((S,), jnp.int32),
            pltpu.VMEM((S,), jnp.int32),
            pltpu.VMEM((_BINS,), jnp.int32),
        ],
    )
    return f(scores)


def _w_kernel(hsT_ref, wp_ref, w_ref):
    w_ref[...] = jnp.dot(wp_ref[...], hsT_ref[...],
                         preferred_element_type=jnp.float32) * (H ** -0.5)


def _score_kernel(qc_ref, wqT_ref, cos_ref, sin_ref, k_ref, wT_ref, out_ref):
    q = jnp.dot(qc_ref[...], wqT_ref[...], preferred_element_type=jnp.float32)
    c = cos_ref[...]
    s = sin_ref[...]
    wT = wT_ref[...]
    kT = k_ref[...].T
    scale = D ** -0.5
    acc = None
    for h in range(H):
        qh = q[:, h * D:(h + 1) * D]
        x1 = qh[:, :D2]
        x2 = qh[:, D2:RD]
        qh = jnp.concatenate([x1 * c - x2 * s, x1 * s + x2 * c, qh[:, RD:]],
                             axis=1)
        sc = jnp.dot(qh, kT, preferred_element_type=jnp.float32) * scale
        sc = jnp.maximum(sc, 0.0) * wT[h, :][:, None]
        acc = sc if acc is None else acc + sc
    out_ref[...] = acc


@jax.jit
def _index_scores(k, hsT, q_compressed, cos, sin, WqT, Wp):
    qc = q_compressed[0]
    wT = pl.pallas_call(
        _w_kernel,
        out_shape=jax.ShapeDtypeStruct((H, S), jnp.float32),
    )(hsT, Wp)

    SB = 256
    grid = (S // SB,)
    out = pl.pallas_call(
        _score_kernel,
        grid=grid,
        in_specs=[
            pl.BlockSpec((SB, QLR), lambda i: (i, 0)),
            pl.BlockSpec((QLR, H * D), lambda i: (0, 0)),
            pl.BlockSpec((SB, D2), lambda i: (i, 0)),
            pl.BlockSpec((SB, D2), lambda i: (i, 0)),
            pl.BlockSpec((S, D), lambda i: (0, 0)),
            pl.BlockSpec((H, SB), lambda i: (0, i)),
        ],
        out_specs=pl.BlockSpec((SB, S), lambda i: (i, 0)),
        out_shape=jax.ShapeDtypeStruct((S, S), jnp.float32),
    )(qc, WqT, cos, sin, k, wT)
    return out


def kernel(hidden_states, q_compressed, freqs_cis, Wq_b, Wk, gamma, beta, Wp):
    cos = freqs_cis[..., 0]
    sin = freqs_cis[..., 1]
    k = hidden_states @ Wk.T
    mu = jnp.mean(k, axis=-1, keepdims=True)
    var = jnp.mean((k - mu) ** 2, axis=-1, keepdims=True)
    k = (k - mu) / jnp.sqrt(var + 1e-5) * gamma + beta
    d2 = RD // 2
    x1 = k[..., :d2]
    x2 = k[..., d2:RD]
    c = cos[None]
    s = sin[None]
    k = jnp.concatenate([x1 * c - x2 * s, x1 * s + x2 * c, k[..., RD:]],
                        axis=-1)
    hsT = hidden_states[0].T
    scores = _index_scores(k[0], hsT, q_compressed, cos, sin, Wq_b.T, Wp)
    return _sc_argsort_desc(scores)[None]


# speedup vs baseline: 1.0094x; 1.0094x over previous
import functools

import jax
import jax.numpy as jnp
from jax import lax
from jax.experimental import pallas as pl
from jax.experimental.pallas import tpu as pltpu
from jax.experimental.pallas import tpu_sc as plsc

B, S, HID, QLR, H, D, RD, TOPK = 1, 2048, 2048, 1536, 16, 128, 64, 2048
D2 = RD // 2


_NW = 32
_RPW = S // _NW
_NV = S // 16
_BINS = 256


def _sc_sort_kernel(scores_hbm, out_hbm, fbuf, ka, va, kb, vb, hist):
    wid = lax.axis_index("s") * 2 + lax.axis_index("c")

    def do_row(i, _):
        row = wid * _RPW + i
        pltpu.sync_copy(scores_hbm.at[row], fbuf)

        def xform(j, _):
            x = fbuf[pl.ds(j * 16, 16)]
            xb = plsc.bitcast(x, jnp.int32)
            xb = jnp.where(x == 0.0, 0, xb)
            key = jnp.where(xb < 0, xb, xb ^ 0x7FFFFFFF)
            ka[pl.ds(j * 16, 16)] = key
            va[pl.ds(j * 16, 16)] = lax.iota(jnp.int32, 16) + j * 16
            return 0

        lax.fori_loop(0, _NV, xform, 0, unroll=4)

        for p, (sk, sv, dk, dv) in enumerate(
                [(ka, va, kb, vb), (kb, vb, ka, va)] * 2):
            shift = 8 * p

            def zero(j, _):
                hist[pl.ds(j * 16, 16)] = jnp.zeros((16,), jnp.int32)
                return 0

            lax.fori_loop(0, _BINS // 16, zero, 0)

            def count(j, _, sk=sk, shift=shift):
                key = sk[pl.ds(j * 16, 16)]
                dgt = lax.shift_right_logical(key, shift) & 0xFF
                occ, last = plsc.scan_count(dgt)
                plsc.addupdate_scatter(hist, [dgt], occ, mask=last)
                return 0

            lax.fori_loop(0, _NV, count, 0, unroll=4)

            carry = jnp.int32(0)
            for j in range(_BINS // 16):
                c = hist[pl.ds(j * 16, 16)]
                incl = plsc.cumsum(c)
                if isinstance(incl, tuple):
                    incl = incl[0]
                hist[pl.ds(j * 16, 16)] = incl - c + carry
                carry = carry + jnp.sum(c, axis=0)

            def permute(j, _, sk=sk, sv=sv, dk=dk, dv=dv, shift=shift):
                key = sk[pl.ds(j * 16, 16)]
                val = sv[pl.ds(j * 16, 16)]
                dgt = lax.shift_right_logical(key, shift) & 0xFF
                occ, last = plsc.scan_count(dgt)
                base = plsc.load_gather(hist, [dgt])
                rank = base + occ - 1
                plsc.store_scatter(dk, [rank], key)
                plsc.store_scatter(dv, [rank], val)
                plsc.store_scatter(hist, [dgt], base + occ, mask=last)
                return 0

            lax.fori_loop(0, _NV, permute, 0, unroll=4)

        pltpu.sync_copy(va, out_hbm.at[row])
        return 0

    lax.fori_loop(0, _RPW, do_row, 0)


@jax.jit
def _sc_argsort_desc(scores):
    mesh = plsc.VectorSubcoreMesh(core_axis_name="c", subcore_axis_name="s")
    f = pl.kernel(
        _sc_sort_kernel,
        mesh=mesh,
        compiler_params=pltpu.CompilerParams(needs_layout_passes=False),
        out_type=jax.ShapeDtypeStruct((S, S), jnp.int32),
        scratch_types=[
            pltpu.VMEM((S,), jnp.float32),
            pltpu.VMEM((S,), jnp.int32),
            pltpu.VMEM((S,), jnp.int32),
            pltpu.VMEM((S,), jnp.int32),
            pltpu.VMEM((S,), jnp.int32),
            pltpu.VMEM((_BINS,), jnp.int32),
        ],
    )
    return f(scores)


def _w_kernel(hsT_ref, wp_ref, w_ref):
    w_ref[...] = jnp.dot(wp_ref[...], hsT_ref[...],
                         preferred_element_type=jnp.float32) * (H ** -0.5)


def _score_kernel(qc_ref, wqT_ref, cos_ref, sin_ref, k_ref, wT_ref, out_ref):
    q = jnp.dot(qc_ref[...], wqT_ref[...], preferred_element_type=jnp.float32)
    c = cos_ref[...]
    s = sin_ref[...]
    wT = wT_ref[...]
    kT = k_ref[...].T
    scale = D ** -0.5
    acc = None
    for h in range(H):
        qh = q[:, h * D:(h + 1) * D]
        x1 = qh[:, :D2]
        x2 = qh[:, D2:RD]
        qh = jnp.concatenate([x1 * c - x2 * s, x1 * s + x2 * c, qh[:, RD:]],
                             axis=1)
        sc = jnp.dot(qh, kT, preferred_element_type=jnp.float32) * scale
        sc = jnp.maximum(sc, 0.0) * wT[h, :][:, None]
        acc = sc if acc is None else acc + sc
    out_ref[...] = acc


@jax.jit
def _index_scores(k, hsT, q_compressed, cos, sin, WqT, Wp):
    qc = q_compressed[0]
    wT = pl.pallas_call(
        _w_kernel,
        out_shape=jax.ShapeDtypeStruct((H, S), jnp.float32),
    )(hsT, Wp)

    SB = 256
    grid = (S // SB,)
    out = pl.pallas_call(
        _score_kernel,
        grid=grid,
        in_specs=[
            pl.BlockSpec((SB, QLR), lambda i: (i, 0)),
            pl.BlockSpec((QLR, H * D), lambda i: (0, 0)),
            pl.BlockSpec((SB, D2), lambda i: (i, 0)),
            pl.BlockSpec((SB, D2), lambda i: (i, 0)),
            pl.BlockSpec((S, D), lambda i: (0, 0)),
            pl.BlockSpec((H, SB), lambda i: (0, i)),
        ],
        out_specs=pl.BlockSpec((SB, S), lambda i: (i, 0)),
        out_shape=jax.ShapeDtypeStruct((S, S), jnp.float32),
    )(qc, WqT, cos, sin, k, wT)
    return out


def kernel(hidden_states, q_compressed, freqs_cis, Wq_b, Wk, gamma, beta, Wp):
    cos = freqs_cis[..., 0]
    sin = freqs_cis[..., 1]
    k = hidden_states @ Wk.T
    mu = jnp.mean(k, axis=-1, keepdims=True)
    var = jnp.mean((k - mu) ** 2, axis=-1, keepdims=True)
    k = (k - mu) / jnp.sqrt(var + 1e-5) * gamma + beta
    d2 = RD // 2
    x1 = k[..., :d2]
    x2 = k[..., d2:RD]
    c = cos[None]
    s = sin[None]
    k = jnp.concatenate([x1 * c - x2 * s, x1 * s + x2 * c, k[..., RD:]],
                        axis=-1)
    hsT = hidden_states[0].T
    scores = _index_scores(k[0], hsT, q_compressed, cos, sin, Wq_b.T, Wp)
    return _sc_argsort_desc(scores)[None]
